# baseline (speedup 1.0000x reference)
import jax
import jax.numpy as jnp
from jax import lax
from jax.experimental import pallas as pl
from jax.experimental.pallas import tpu as pltpu
from jax.experimental.pallas import tpu_sc as plsc

B = 8
N = 1024
H = 64
HH = H * H
K = 16
S = B * N
R = B * HH
NW = 32
RPW = R // NW
CR = 32
NCHUNK = RPW // CR
NGROUP = NCHUNK // 2


def _full16(v):
    return jnp.full((16,), v, jnp.int32)


NBUF = 3


def _sc_body(m_hbm, kv_hbm, nvec_hbm, wvec_hbm, gvec_hbm, out_hbm,
             buf0, buf1, buf2, nvec_v, wvec_v, gidx_v, kvrows,
             is0, is1, is2, os0, os1, os2, gsem):
    wid = lax.axis_index("c") * 16 + lax.axis_index("s")
    row0 = wid * RPW
    iota16 = lax.broadcasted_iota(jnp.int32, (16,), 0)
    bufs = (buf0, buf1, buf2)
    isems = (is0, is1, is2)
    osems = (os0, os1, os2)

    def in_copy(g):
        s = g % NBUF
        return pltpu.make_async_copy(
            m_hbm.at[pl.ds(row0 + g * CR, CR), :], bufs[s], isems[s])

    def out_copy(g):
        s = g % NBUF
        return pltpu.make_async_copy(
            bufs[s], out_hbm.at[pl.ds(row0 + g * CR, CR), :], osems[s])

    in_copy(0).start()
    in_copy(1).start()

    pltpu.sync_copy(nvec_hbm.at[wid], nvec_v)
    pltpu.sync_copy(wvec_hbm.at[wid], wvec_v)
    pltpu.sync_copy(gvec_hbm.at[wid], gidx_v)
    pltpu.async_copy(kv_hbm.at[gidx_v], kvrows, gsem).wait()
    nv = nvec_v[...]
    wv = wvec_v[...]
    lane_on = wv > 0.0

    def apply_updates(g, buf):
        x0 = row0 + g * CR

        def row_body(rl, carry):
            x = x0 + rl
            r = jnp.bitwise_and(lax.shift_right_logical(x, 6), H - 1)
            c = jnp.bitwise_and(x, H - 1)
            mkv = plsc.load_gather(kvrows, [iota16, _full16(r)])
            mvv = plsc.load_gather(kvrows, [iota16, _full16(H + c)])
            plsc.addupdate_scatter(buf, [_full16(rl), nv], mkv * wv * mvv,
                                   mask=lane_on)
            return carry

        lax.fori_loop(0, CR, row_body, 0)

    for h in range(NCHUNK):
        if h + 1 < NCHUNK:
            if h >= 2:
                out_copy(h - 2).wait()
            in_copy(h + 1).start()
        in_copy(h).wait()
        apply_updates(h, bufs[h % NBUF])
        out_copy(h).start()
    for g in range(NCHUNK - 3, NCHUNK):
        out_copy(g).wait()


@jax.jit
def kernel(M, M_k, M_v, indices_update):
    idx = indices_update.astype(jnp.int32)
    eq = idx[:, :, None] == idx[:, None, :]
    first = ~jnp.tril(eq, k=-1).any(-1)
    cnt = eq.sum(-1)
    wrow = jnp.where(first, cnt, 0).astype(jnp.float32)
    nrow = idx
    grow = jnp.arange(B, dtype=jnp.int32)[:, None] * N + idx

    b_of_w = jnp.arange(NW) // (NW // B)
    nvec = nrow[b_of_w]
    wvec = wrow[b_of_w]
    gvec = grow[b_of_w]

    m2 = M.transpose(0, 2, 3, 1).reshape(R, N)
    kv = jnp.concatenate([M_k.reshape(S, H), M_v.reshape(S, H)], axis=-1)

    sc_kernel = pl.kernel(
        _sc_body,
        out_type=jax.ShapeDtypeStruct((R, N), jnp.float32),
        mesh=plsc.VectorSubcoreMesh(core_axis_name="c", subcore_axis_name="s"),
        scratch_types=[
            pltpu.VMEM((CR, N), jnp.float32),
            pltpu.VMEM((CR, N), jnp.float32),
            pltpu.VMEM((CR, N), jnp.float32),
            pltpu.VMEM((K,), jnp.int32),
            pltpu.VMEM((K,), jnp.float32),
            pltpu.VMEM((K,), jnp.int32),
            pltpu.VMEM((K, 2 * H), jnp.float32),
            pltpu.SemaphoreType.DMA,
            pltpu.SemaphoreType.DMA,
            pltpu.SemaphoreType.DMA,
            pltpu.SemaphoreType.DMA,
            pltpu.SemaphoreType.DMA,
            pltpu.SemaphoreType.DMA,
            pltpu.SemaphoreType.DMA,
        ],
        compiler_params=pltpu.CompilerParams(needs_layout_passes=False),
    )
    out2 = sc_kernel(m2, kv, nvec, wvec, gvec)
    return out2.reshape(B, H, H, N).transpose(0, 3, 1, 2)


# speedup vs baseline: 1.0271x; 1.0271x over previous
import jax
import jax.numpy as jnp
from jax import lax
from jax.experimental import pallas as pl
from jax.experimental.pallas import tpu as pltpu
from jax.experimental.pallas import tpu_sc as plsc

B = 8
N = 1024
H = 64
HH = H * H
K = 16
S = B * N
R = B * HH
NW = 32
RPW = R // NW
CR = 32
NCHUNK = RPW // CR
NGROUP = NCHUNK // 2


def _full16(v):
    return jnp.full((16,), v, jnp.int32)


def _sc_body(m_hbm, kv_hbm, nvec_hbm, wvec_hbm, gvec_hbm, out_hbm,
             buf0, buf1, nvec_v, wvec_v, gidx_v, kvrows,
             is0, is1, os0, os1, gsem):
    wid = lax.axis_index("c") * 16 + lax.axis_index("s")
    row0 = wid * RPW
    iota16 = lax.broadcasted_iota(jnp.int32, (16,), 0)

    def in_copy(g, buf, sem):
        return pltpu.make_async_copy(
            m_hbm.at[pl.ds(row0 + g * CR, CR), :], buf, sem)

    def out_copy(g, buf, sem):
        return pltpu.make_async_copy(
            buf, out_hbm.at[pl.ds(row0 + g * CR, CR), :], sem)

    in_copy(0, buf0, is0).start()
    in_copy(1, buf1, is1).start()

    pltpu.sync_copy(nvec_hbm.at[wid], nvec_v)
    pltpu.sync_copy(wvec_hbm.at[wid], wvec_v)
    pltpu.sync_copy(gvec_hbm.at[wid], gidx_v)
    pltpu.async_copy(kv_hbm.at[gidx_v], kvrows, gsem).wait()
    nv = nvec_v[...]
    wv = wvec_v[...]
    lane_on = wv > 0.0

    def apply_updates(g, buf):
        x0 = row0 + g * CR
        r = jnp.bitwise_and(lax.shift_right_logical(x0, 6), H - 1)
        c0 = jnp.bitwise_and(x0, H - 1)
        mkw = plsc.load_gather(kvrows, [iota16, _full16(r)]) * wv

        def row_body(rl, carry):
            mvv = plsc.load_gather(kvrows, [iota16, _full16(H + c0 + rl)])
            plsc.addupdate_scatter(buf, [_full16(rl), nv], mkw * mvv,
                                   mask=lane_on)
            return carry

        lax.fori_loop(0, CR, row_body, 0)

    def group(go, carry):
        for s, buf, isem, osem in ((0, buf0, is0, os0), (1, buf1, is1, os1)):
            g = 2 * go + s
            in_copy(g, buf, isem).wait()
            apply_updates(g, buf)
            oc = out_copy(g, buf, osem)
            oc.start()
            oc.wait()

            @pl.when(g + 2 < NCHUNK)
            def _():
                in_copy(g + 2, buf, isem).start()
        return carry

    lax.fori_loop(0, NGROUP, group, 0)


@jax.jit
def kernel(M, M_k, M_v, indices_update):
    idx = indices_update.astype(jnp.int32)
    eq = idx[:, :, None] == idx[:, None, :]
    first = ~jnp.tril(eq, k=-1).any(-1)
    cnt = eq.sum(-1)
    wrow = jnp.where(first, cnt, 0).astype(jnp.float32)
    nrow = idx
    grow = jnp.arange(B, dtype=jnp.int32)[:, None] * N + idx

    b_of_w = jnp.arange(NW) // (NW // B)
    nvec = nrow[b_of_w]
    wvec = wrow[b_of_w]
    gvec = grow[b_of_w]

    m2 = M.transpose(0, 2, 3, 1).reshape(R, N)
    kv = jnp.concatenate([M_k.reshape(S, H), M_v.reshape(S, H)], axis=-1)

    sc_kernel = pl.kernel(
        _sc_body,
        out_type=jax.ShapeDtypeStruct((R, N), jnp.float32),
        mesh=plsc.VectorSubcoreMesh(core_axis_name="c", subcore_axis_name="s"),
        scratch_types=[
            pltpu.VMEM((CR, N), jnp.float32),
            pltpu.VMEM((CR, N), jnp.float32),
            pltpu.VMEM((K,), jnp.int32),
            pltpu.VMEM((K,), jnp.float32),
            pltpu.VMEM((K,), jnp.int32),
            pltpu.VMEM((K, 2 * H), jnp.float32),
            pltpu.SemaphoreType.DMA,
            pltpu.SemaphoreType.DMA,
            pltpu.SemaphoreType.DMA,
            pltpu.SemaphoreType.DMA,
            pltpu.SemaphoreType.DMA,
        ],
        compiler_params=pltpu.CompilerParams(needs_layout_passes=False),
    )
    out2 = sc_kernel(m2, kv, nvec, wvec, gvec)
    return out2.reshape(B, H, H, N).transpose(0, 3, 1, 2)
